# Initial kernel scaffold; baseline (speedup 1.0000x reference)
import jax
import jax.numpy as jnp
from jax.experimental import pallas as pl


def kernel(x, weight):
    raise NotImplementedError("write your pallas kernel here")


# speedup vs baseline: 1.8553x; 1.8553x over previous
import functools

import jax
import jax.numpy as jnp
from jax import lax
from jax.experimental import pallas as pl
from jax.experimental.pallas import tpu as pltpu
from jax.experimental.pallas import tpu_sc as plsc

VOCAB = 1000000
D = 64
B = 16384 * 50
NC = 2
NS = 16
NW = NC * NS
B_PER_W = B // NW
CHUNK = 128
GROUP = 8
CHUNKS_PER_W = B_PER_W // CHUNK
GROUPS = CHUNKS_PER_W // GROUP


def _emb_body(x_hbm, table_hbm, out_hbm, idx_v, rows_v, sem):
    wid = lax.axis_index("s") * NC + lax.axis_index("c")
    row_base = wid * CHUNKS_PER_W
    pltpu.sync_copy(x_hbm.at[pl.ds(row_base, CHUNKS_PER_W)], idx_v)

    def group_body(g, _):
        copies = []
        for j in range(GROUP):
            copies.append(
                pltpu.async_copy(
                    table_hbm.at[idx_v.at[g * GROUP + j]],
                    rows_v.at[j],
                    sem,
                )
            )
        for c in copies:
            c.wait()
        pltpu.sync_copy(rows_v, out_hbm.at[pl.ds(row_base + g * GROUP, GROUP)])
        return ()

    lax.fori_loop(0, GROUPS, group_body, (), unroll=False)


@jax.jit
def _emb_call(x_flat, weight):
    mesh = plsc.VectorSubcoreMesh(core_axis_name="c", subcore_axis_name="s")
    return pl.kernel(
        _emb_body,
        out_type=jax.ShapeDtypeStruct((B // CHUNK, CHUNK, D), jnp.float32),
        mesh=mesh,
        scratch_types=[
            pltpu.VMEM((CHUNKS_PER_W, CHUNK), jnp.int32),
            pltpu.VMEM((GROUP, CHUNK, D), jnp.float32),
            pltpu.SemaphoreType.DMA,
        ],
        compiler_params=pltpu.CompilerParams(use_tc_tiling_on_sc=False),
    )(x_flat, weight)


def kernel(x, weight):
    x_flat = x.reshape(B // CHUNK, CHUNK).astype(jnp.int32)
    out = _emb_call(x_flat, weight)
    return out.reshape(x.shape[0], x.shape[1], D)


# speedup vs baseline: 1.8710x; 1.0085x over previous
import functools

import jax
import jax.numpy as jnp
from jax import lax
from jax.experimental import pallas as pl
from jax.experimental.pallas import tpu as pltpu
from jax.experimental.pallas import tpu_sc as plsc

VOCAB = 1000000
D = 64
B = 16384 * 50
NC = 2
NS = 16
NW = NC * NS
B_PER_W = B // NW
CHUNK = 128
GROUP = 4
CHUNKS_PER_W = B_PER_W // CHUNK
GROUPS = CHUNKS_PER_W // GROUP


def _fire(table_hbm, idx_v, rows_buf, sem, g):
    copies = []
    for j in range(GROUP):
        copies.append(
            pltpu.async_copy(
                table_hbm.at[idx_v.at[g * GROUP + j]],
                rows_buf.at[j],
                sem,
            )
        )
    return copies


def _drain(table_hbm, idx_v, rows_buf, sem):
    for j in range(GROUP):
        pltpu.make_async_copy(
            table_hbm.at[idx_v.at[j]],
            rows_buf.at[j],
            sem,
        ).wait()


def _emb_body(x_hbm, table_hbm, out_hbm, idx_v, rows0, rows1, sem0, sem1):
    wid = lax.axis_index("s") * NC + lax.axis_index("c")
    row_base = wid * CHUNKS_PER_W
    pltpu.sync_copy(x_hbm.at[pl.ds(row_base, CHUNKS_PER_W)], idx_v)

    def store(rows_buf, g):
        pltpu.sync_copy(rows_buf, out_hbm.at[pl.ds(row_base + g * GROUP, GROUP)])

    _fire(table_hbm, idx_v, rows0, sem0, 0)

    def pair_body(i, _):
        g = 2 * i
        _drain(table_hbm, idx_v, rows0, sem0)
        _fire(table_hbm, idx_v, rows1, sem1, g + 1)
        store(rows0, g)
        _drain(table_hbm, idx_v, rows1, sem1)
        _fire(table_hbm, idx_v, rows0, sem0, g + 2)
        store(rows1, g + 1)
        return ()

    lax.fori_loop(0, GROUPS // 2 - 1, pair_body, (), unroll=False)

    g = GROUPS - 2
    _drain(table_hbm, idx_v, rows0, sem0)
    _fire(table_hbm, idx_v, rows1, sem1, g + 1)
    store(rows0, g)
    _drain(table_hbm, idx_v, rows1, sem1)
    store(rows1, g + 1)


@jax.jit
def _emb_call(x_flat, weight):
    mesh = plsc.VectorSubcoreMesh(core_axis_name="c", subcore_axis_name="s")
    return pl.kernel(
        _emb_body,
        out_type=jax.ShapeDtypeStruct((B // CHUNK, CHUNK, D), jnp.float32),
        mesh=mesh,
        scratch_types=[
            pltpu.VMEM((CHUNKS_PER_W, CHUNK), jnp.int32),
            pltpu.VMEM((GROUP, CHUNK, D), jnp.float32),
            pltpu.VMEM((GROUP, CHUNK, D), jnp.float32),
            pltpu.SemaphoreType.DMA,
            pltpu.SemaphoreType.DMA,
        ],
        compiler_params=pltpu.CompilerParams(use_tc_tiling_on_sc=False),
    )(x_flat, weight)


def kernel(x, weight):
    x_flat = x.reshape(B // CHUNK, CHUNK).astype(jnp.int32)
    out = _emb_call(x_flat, weight)
    return out.reshape(x.shape[0], x.shape[1], D)
